# Initial kernel scaffold; baseline (speedup 1.0000x reference)
import jax
import jax.numpy as jnp
from jax.experimental import pallas as pl


def kernel(atom_features, pair_features, pair_split, atom_split, atom_to_pair, params):
    raise NotImplementedError("write your pallas kernel here")


# speedup vs baseline: 4.5156x; 4.5156x over previous
import functools

import jax
import jax.numpy as jnp
from jax import lax
from jax.experimental import pallas as pl
from jax.experimental.pallas import tpu as pltpu
from jax.experimental.pallas import tpu_sc as plsc

F32 = jnp.float32


def _atom_pre_body(x, waa, baa, wg, bg, aa_o, g_o):
    xv = x[...]
    aa_o[...] = jnp.tanh(jnp.dot(xv, waa[...], preferred_element_type=F32) + baa[...])
    g_o[...] = jnp.dot(xv, wg[...], preferred_element_type=F32) + bg[...]


def _pair_pre_body(x, wpa, bpa, wpp, bpp, pa_o, pp_o):
    xv = x[...]
    pa_o[...] = jnp.tanh(jnp.dot(xv, wpa[...], preferred_element_type=F32) + bpa[...])
    pp_o[...] = jnp.tanh(jnp.dot(xv, wpp[...], preferred_element_type=F32) + bpp[...])


def _mix_body(aa, ps0, ps1, wa, wb, b, a_o):
    s = ps0[...] + ps1[...]
    a_o[...] = jnp.tanh(jnp.dot(aa[...], wa[...], preferred_element_type=F32)
                        + jnp.dot(s, wb[...], preferred_element_type=F32) + b[...])


def _pair_post_body(t, pp, x, wpa_t, wpb, bp, w2a, w2b, b2, pa2_o):
    p1 = jnp.tanh(jnp.dot(t[...], wpa_t[...], preferred_element_type=F32)
                  + jnp.dot(pp[...], wpb[...], preferred_element_type=F32) + bp[...])
    pa2_o[...] = jnp.tanh(jnp.dot(x[...], w2a[...], preferred_element_type=F32)
                          + jnp.dot(p1, w2b[...], preferred_element_type=F32) + b2[...])


def _atom_final_body(x, a1, ps0, ps1, waa2a, waa2b, baa2, wa2a, wa2b, ba2,
                     wda, wdb, wdc, bd, scale, shift, h_o):
    xv = x[...]
    a1v = a1[...]
    aa2 = jnp.tanh(jnp.dot(xv, waa2a[...], preferred_element_type=F32)
                   + jnp.dot(a1v, waa2b[...], preferred_element_type=F32) + baa2[...])
    pasum2 = ps0[...] + ps1[...]
    a2 = jnp.tanh(jnp.dot(aa2, wa2a[...], preferred_element_type=F32)
                  + jnp.dot(pasum2, wa2b[...], preferred_element_type=F32) + ba2[...])
    z = jnp.tanh(jnp.dot(xv, wda[...], preferred_element_type=F32)
                 + jnp.dot(a1v, wdb[...], preferred_element_type=F32)
                 + jnp.dot(a2, wdc[...], preferred_element_type=F32) + bd[...])
    h_o[...] = z * scale[...] + shift[...]


def _row_spec(r, d):
    return pl.BlockSpec((r, d), lambda i: (i, 0))


def _full_spec(shape):
    return pl.BlockSpec(shape, lambda i: tuple(0 for _ in shape))


def _tc_call(body, n_rows, block_rows, xs, weights, out_dims):
    grid = (n_rows // block_rows,)
    in_specs = ([_row_spec(block_rows, x.shape[1]) for x in xs]
                + [_full_spec(w.shape) for w in weights])
    out_specs = [_row_spec(block_rows, d) for d in out_dims]
    out_shape = [jax.ShapeDtypeStruct((n_rows, d), F32) for d in out_dims]
    outs = pl.pallas_call(
        body, grid=grid, in_specs=in_specs, out_specs=out_specs,
        out_shape=out_shape,
    )(*xs, *weights)
    return outs


def _sc_tanh(x):
    return 1.0 - 2.0 / (jnp.exp(2.0 * x) + 1.0)


def _make_segsum(M, N, F):
    info = plsc.get_sparse_core_info()
    NC, NS = info.num_cores, info.num_subcores
    NW = NC * NS
    per_tile = M // NW
    CI = 125
    KI = 8
    CHUNK = CI * KI
    n_chunks = per_tile // CHUNK
    assert per_tile % CHUNK == 0 and per_tile % 8 == 0
    ZR = 1000
    n_zch = N // ZR
    assert N % ZR == 0
    zrounds = (n_zch + NS - 1) // NS
    mesh = plsc.VectorSubcoreMesh(core_axis_name="c", subcore_axis_name="s")

    @functools.partial(
        pl.kernel,
        out_type=jax.ShapeDtypeStruct((NC, N, F), F32),
        mesh=mesh,
        compiler_params=pltpu.CompilerParams(use_tc_tiling_on_sc=False),
        scratch_types=[
            pltpu.VMEM((KI, CI), jnp.int32),
            pltpu.VMEM((CHUNK, F), F32),
            pltpu.VMEM((ZR, F), F32),
            pltpu.VMEM_SHARED((N, F), F32),
        ],
    )
    def segsum(pa_hbm, split_hbm, out_hbm, idx_v, pa_v, zbuf, table):
        c = lax.axis_index("c")
        s = lax.axis_index("s")
        wid = s * NC + c

        def zb(r, carry):
            zbuf[r, :] = jnp.zeros((16,), F32)
            return carry
        lax.fori_loop(0, ZR, zb, None)
        for k in range(zrounds):
            cid = s + k * NS
            @pl.when(cid < n_zch)
            def _():
                pltpu.sync_copy(zbuf, table.at[pl.ds(cid * ZR, ZR)])
        plsc.subcore_barrier()

        base = wid * per_tile

        def chunk_body(ci, carry):
            off = base + ci * CHUNK
            pltpu.sync_copy(split_hbm.at[pl.ds(off // CI, KI)], idx_v)
            pltpu.sync_copy(pa_hbm.at[pl.ds(off, CHUNK)], pa_v)
            for k in range(KI):
                pltpu.sync_copy(pa_v.at[pl.ds(k * CI, CI)],
                                table.at[idx_v.at[k]], add=True)
            return carry
        lax.fori_loop(0, n_chunks, chunk_body, None)
        plsc.subcore_barrier()

        for k in range(zrounds):
            cid = s + k * NS
            @pl.when(cid < n_zch)
            def _():
                pltpu.sync_copy(table.at[pl.ds(cid * ZR, ZR)], zbuf)
                pltpu.sync_copy(zbuf, out_hbm.at[c, pl.ds(cid * ZR, ZR)])

    return segsum


def _make_gather_combine(M, N, F):
    info = plsc.get_sparse_core_info()
    NC, NS = info.num_cores, info.num_subcores
    NW = NC * NS
    per_tile = M // NW
    CP = 1000
    ROWS = 2 * CP
    n_chunks = per_tile // CP
    assert per_tile % CP == 0
    pieces = []
    o = 0
    while o < ROWS:
        sz = min(128, ROWS - o)
        pieces.append((o, sz))
        o += sz
    mesh = plsc.VectorSubcoreMesh(core_axis_name="c", subcore_axis_name="s")

    @functools.partial(
        pl.kernel,
        out_type=jax.ShapeDtypeStruct((M, F), F32),
        mesh=mesh,
        compiler_params=pltpu.CompilerParams(use_tc_tiling_on_sc=False),
        scratch_types=[
            pltpu.VMEM((ROWS,), jnp.int32),
            pltpu.VMEM((ROWS, 2 * F), F32),
            pltpu.VMEM((CP, F), F32),
            pltpu.SemaphoreType.DMA,
        ],
    )
    def gather_combine(g_hbm, idx_hbm, t_hbm, idx_v, rows_v, t_v, sem):
        c = lax.axis_index("c")
        s = lax.axis_index("s")
        wid = s * NC + c
        pbase = wid * per_tile

        def chunk_body(ci, carry):
            p0 = pbase + ci * CP
            pltpu.sync_copy(idx_hbm.at[pl.ds(2 * p0, ROWS)], idx_v)
            for (o, sz) in pieces:
                pltpu.async_copy(g_hbm.at[idx_v.at[pl.ds(o, sz)]],
                                 rows_v.at[pl.ds(o, sz)], sem).wait()

            def pair_body(p, carry2):
                u = rows_v[2 * p, pl.ds(0, F)] + rows_v[2 * p + 1, pl.ds(F, F)]
                v = rows_v[2 * p + 1, pl.ds(0, F)] + rows_v[2 * p, pl.ds(F, F)]
                t_v[p, :] = _sc_tanh(u) + _sc_tanh(v)
                return carry2
            lax.fori_loop(0, CP, pair_body, None)
            pltpu.sync_copy(t_v, t_hbm.at[pl.ds(p0, CP)])
            return carry
        lax.fori_loop(0, n_chunks, chunk_body, None)

    return gather_combine


def kernel(atom_features, pair_features, pair_split, atom_split, atom_to_pair, params):
    L1, L2 = params['layers']
    N, AD = atom_features.shape
    M, PD = pair_features.shape
    F = L1['W_AA'].shape[1]
    GF = params['W_dense'].shape[1]

    row = lambda b: b[None, :]
    Wg = jnp.concatenate([L1['W_AP'][:AD], L1['W_AP'][AD:]], axis=1)
    bg = jnp.concatenate([L1['b_AP'], jnp.zeros_like(L1['b_AP'])])[None, :]
    Wd = params['W_dense']
    scale = params['bn_gamma'] / jnp.sqrt(params['bn_var'] + 1e-3)
    shift = params['bn_beta'] - params['bn_mean'] * scale

    idx_flat = atom_to_pair.reshape(-1)
    split2d = pair_split.reshape(M // 125, 125)

    R_A, R_P = 5000, 8000

    AA1, G = _tc_call(_atom_pre_body, N, R_A, [atom_features],
                      [L1['W_AA'], row(L1['b_AA']), Wg, bg], [F, 2 * F])

    PA1, PP1 = _tc_call(_pair_pre_body, M, R_P, [pair_features],
                        [L1['W_PA'], row(L1['b_PA']), L1['W_PP'], row(L1['b_PP'])],
                        [F, F])

    ps1 = _make_segsum(M, N, F)(PA1, split2d)
    t1 = _make_gather_combine(M, N, F)(G, idx_flat)

    (A1,) = _tc_call(_mix_body, N, R_A, [AA1, ps1[0], ps1[1]],
                     [L1['W_A'][:F], L1['W_A'][F:], row(L1['b_A'])], [F])

    (PA2,) = _tc_call(_pair_post_body, M, R_P, [t1, PP1, pair_features],
                      [L1['W_P'][:F], L1['W_P'][F:], row(L1['b_P']),
                       L2['W_PA'][:PD], L2['W_PA'][PD:], row(L2['b_PA'])], [F])

    ps2 = _make_segsum(M, N, F)(PA2, split2d)

    (h,) = _tc_call(_atom_final_body, N, R_A,
                    [atom_features, A1, ps2[0], ps2[1]],
                    [L2['W_AA'][:AD], L2['W_AA'][AD:], row(L2['b_AA']),
                     L2['W_A'][:F], L2['W_A'][F:], row(L2['b_A']),
                     Wd[:AD], Wd[AD:AD + F], Wd[AD + F:], row(params['b_dense']),
                     row(scale), row(shift)], [GF])
    return h


# speedup vs baseline: 7.0908x; 1.5703x over previous
import functools

import jax
import jax.numpy as jnp
from jax import lax
from jax.experimental import pallas as pl
from jax.experimental.pallas import tpu as pltpu
from jax.experimental.pallas import tpu_sc as plsc

F32 = jnp.float32


def _atom_pre_body(x, waa, baa, wg, bg, aa_o, g_o):
    xv = x[...]
    aa_o[...] = jnp.tanh(jnp.dot(xv, waa[...], preferred_element_type=F32) + baa[...])
    g_o[...] = jnp.dot(xv, wg[...], preferred_element_type=F32) + bg[...]


def _pair_pre_body(x, wpa, bpa, wpp, bpp, pa_o, pp_o):
    xv = x[...]
    pa_o[...] = jnp.tanh(jnp.dot(xv, wpa[...], preferred_element_type=F32) + bpa[...])
    pp_o[...] = jnp.tanh(jnp.dot(xv, wpp[...], preferred_element_type=F32) + bpp[...])


def _mix_body(aa, ps0, ps1, wa, wb, b, a_o):
    s = ps0[...] + ps1[...]
    a_o[...] = jnp.tanh(jnp.dot(aa[...], wa[...], preferred_element_type=F32)
                        + jnp.dot(s, wb[...], preferred_element_type=F32) + b[...])


def _pair_post_body(t, pp, x, wpa_t, wpb, bp, w2a, w2b, b2, pa2_o):
    p1 = jnp.tanh(jnp.dot(t[...], wpa_t[...], preferred_element_type=F32)
                  + jnp.dot(pp[...], wpb[...], preferred_element_type=F32) + bp[...])
    pa2_o[...] = jnp.tanh(jnp.dot(x[...], w2a[...], preferred_element_type=F32)
                          + jnp.dot(p1, w2b[...], preferred_element_type=F32) + b2[...])


def _atom_final_body(x, a1, ps0, ps1, waa2a, waa2b, baa2, wa2a, wa2b, ba2,
                     wda, wdb, wdc, bd, scale, shift, h_o):
    xv = x[...]
    a1v = a1[...]
    aa2 = jnp.tanh(jnp.dot(xv, waa2a[...], preferred_element_type=F32)
                   + jnp.dot(a1v, waa2b[...], preferred_element_type=F32) + baa2[...])
    pasum2 = ps0[...] + ps1[...]
    a2 = jnp.tanh(jnp.dot(aa2, wa2a[...], preferred_element_type=F32)
                  + jnp.dot(pasum2, wa2b[...], preferred_element_type=F32) + ba2[...])
    z = jnp.tanh(jnp.dot(xv, wda[...], preferred_element_type=F32)
                 + jnp.dot(a1v, wdb[...], preferred_element_type=F32)
                 + jnp.dot(a2, wdc[...], preferred_element_type=F32) + bd[...])
    h_o[...] = z * scale[...] + shift[...]


def _row_spec(r, d):
    return pl.BlockSpec((r, d), lambda i: (i, 0))


def _full_spec(shape):
    return pl.BlockSpec(shape, lambda i: tuple(0 for _ in shape))


def _tc_call(body, n_rows, block_rows, xs, weights, out_dims):
    grid = (n_rows // block_rows,)
    in_specs = ([_row_spec(block_rows, x.shape[1]) for x in xs]
                + [_full_spec(w.shape) for w in weights])
    out_specs = [_row_spec(block_rows, d) for d in out_dims]
    out_shape = [jax.ShapeDtypeStruct((n_rows, d), F32) for d in out_dims]
    outs = pl.pallas_call(
        body, grid=grid, in_specs=in_specs, out_specs=out_specs,
        out_shape=out_shape,
    )(*xs, *weights)
    return outs


def _sc_tanh(x):
    return 1.0 - 2.0 / (jnp.exp(2.0 * x) + 1.0)


def _make_segsum(M, N, F):
    info = plsc.get_sparse_core_info()
    NC, NS = info.num_cores, info.num_subcores
    NW = NC * NS
    per_tile = M // NW
    CI = 125
    KI = 8
    CHUNK = CI * KI
    PK = CHUNK // 8
    n_chunks = per_tile // CHUNK
    assert per_tile % CHUNK == 0 and per_tile % 8 == 0
    ZR = 1000
    n_zch = N // ZR
    assert N % ZR == 0
    zrounds = (n_zch + NS - 1) // NS
    mesh = plsc.VectorSubcoreMesh(core_axis_name="c", subcore_axis_name="s")

    @functools.partial(
        pl.kernel,
        out_type=jax.ShapeDtypeStruct((NC, N, F), F32),
        mesh=mesh,
        compiler_params=pltpu.CompilerParams(use_tc_tiling_on_sc=False),
        scratch_types=[
            pltpu.VMEM((KI, CI), jnp.int32),
            pltpu.VMEM((PK, 8 * F), F32),
            pltpu.VMEM((CHUNK, F), F32),
            pltpu.VMEM((ZR, F), F32),
            pltpu.VMEM_SHARED((N, F), F32),
            pltpu.SemaphoreType.DMA,
        ],
    )
    def segsum(pa_hbm, split_hbm, out_hbm, idx_v, pa_pk, pa_v, zbuf, table, sem):
        c = lax.axis_index("c")
        s = lax.axis_index("s")
        wid = s * NC + c

        def zb(r, carry):
            zbuf[r, :] = jnp.zeros((16,), F32)
            return carry
        lax.fori_loop(0, ZR, zb, None)
        for k in range(zrounds):
            cid = s + k * NS
            @pl.when(cid < n_zch)
            def _():
                pltpu.sync_copy(zbuf, table.at[pl.ds(cid * ZR, ZR)])
        plsc.subcore_barrier()

        base = wid * per_tile

        def chunk_body(ci, carry):
            off = base + ci * CHUNK
            pltpu.sync_copy(split_hbm.at[pl.ds(off // CI, KI)], idx_v)
            pltpu.sync_copy(pa_hbm.at[pl.ds(off // 8, PK)], pa_pk)

            def repack(r, carry2):
                for cc in range(8):
                    pa_v[8 * r + cc, :] = pa_pk[r, pl.ds(16 * cc, 16)]
                return carry2
            lax.fori_loop(0, PK, repack, None)

            descs = []
            for k in range(KI):
                descs.append(pltpu.async_copy(
                    pa_v.at[pl.ds(k * CI, CI)], table.at[idx_v.at[k]], sem,
                    add=True))
            for d in descs:
                d.wait()
            return carry
        lax.fori_loop(0, n_chunks, chunk_body, None)
        plsc.subcore_barrier()

        for k in range(zrounds):
            cid = s + k * NS
            @pl.when(cid < n_zch)
            def _():
                pltpu.sync_copy(table.at[pl.ds(cid * ZR, ZR)], zbuf)
                pltpu.sync_copy(zbuf, out_hbm.at[c, pl.ds(cid * ZR, ZR)])

    return segsum


def _make_gather_combine(M, N, F):
    info = plsc.get_sparse_core_info()
    NC, NS = info.num_cores, info.num_subcores
    NW = NC * NS
    per_tile = M // NW
    CP = 1000
    ROWS = 2 * CP
    PK = CP // 8
    n_chunks = per_tile // CP
    assert per_tile % CP == 0
    pieces = []
    o = 0
    while o < ROWS:
        sz = min(128, ROWS - o)
        pieces.append((o, sz))
        o += sz
    mesh = plsc.VectorSubcoreMesh(core_axis_name="c", subcore_axis_name="s")

    @functools.partial(
        pl.kernel,
        out_type=jax.ShapeDtypeStruct((M // 8, 8 * F), F32),
        mesh=mesh,
        compiler_params=pltpu.CompilerParams(use_tc_tiling_on_sc=False),
        scratch_types=[
            pltpu.VMEM((ROWS,), jnp.int32),
            pltpu.VMEM((ROWS, 2 * F), F32),
            pltpu.VMEM((PK, 8 * F), F32),
            pltpu.SemaphoreType.DMA,
        ],
    )
    def gather_combine(g_hbm, idx_hbm, t_hbm, idx_v, rows_v, t_v, sem):
        c = lax.axis_index("c")
        s = lax.axis_index("s")
        wid = s * NC + c
        pbase = wid * per_tile

        def chunk_body(ci, carry):
            p0 = pbase + ci * CP
            pltpu.sync_copy(idx_hbm.at[pl.ds(2 * p0, ROWS)], idx_v)
            descs = []
            for (o, sz) in pieces:
                descs.append(pltpu.async_copy(
                    g_hbm.at[idx_v.at[pl.ds(o, sz)]],
                    rows_v.at[pl.ds(o, sz)], sem))
            for d in descs:
                d.wait()

            def pair_body(p, carry2):
                u = rows_v[2 * p, pl.ds(0, F)] + rows_v[2 * p + 1, pl.ds(F, F)]
                v = rows_v[2 * p + 1, pl.ds(0, F)] + rows_v[2 * p, pl.ds(F, F)]
                t_v[p // 8, pl.ds((p % 8) * F, F)] = _sc_tanh(u) + _sc_tanh(v)
                return carry2
            lax.fori_loop(0, CP, pair_body, None)
            pltpu.sync_copy(t_v, t_hbm.at[pl.ds(p0 // 8, PK)])
            return carry
        lax.fori_loop(0, n_chunks, chunk_body, None)

    return gather_combine


def kernel(atom_features, pair_features, pair_split, atom_split, atom_to_pair, params):
    L1, L2 = params['layers']
    N, AD = atom_features.shape
    M, PD = pair_features.shape
    F = L1['W_AA'].shape[1]
    GF = params['W_dense'].shape[1]
    MP = M // 8

    row = lambda b: b[None, :]
    bd8 = lambda w: jax.scipy.linalg.block_diag(*([w] * 8))
    t8 = lambda b: jnp.tile(b, 8)[None, :]
    Wg = jnp.concatenate([L1['W_AP'][:AD], L1['W_AP'][AD:]], axis=1)
    bg = jnp.concatenate([L1['b_AP'], jnp.zeros_like(L1['b_AP'])])[None, :]
    Wd = params['W_dense']
    scale = params['bn_gamma'] / jnp.sqrt(params['bn_var'] + 1e-3)
    shift = params['bn_beta'] - params['bn_mean'] * scale

    idx_flat = atom_to_pair.reshape(-1)
    split2d = pair_split.reshape(M // 125, 125)
    Xp = pair_features.reshape(MP, 8 * PD)

    R_A, R_PP = 5000, 2000

    AA1, G = _tc_call(_atom_pre_body, N, R_A, [atom_features],
                      [L1['W_AA'], row(L1['b_AA']), Wg, bg], [F, 2 * F])

    PA1, PP1 = _tc_call(_pair_pre_body, MP, R_PP, [Xp],
                        [bd8(L1['W_PA']), t8(L1['b_PA']),
                         bd8(L1['W_PP']), t8(L1['b_PP'])],
                        [8 * F, 8 * F])

    ps1 = _make_segsum(M, N, F)(PA1, split2d)
    t1 = _make_gather_combine(M, N, F)(G, idx_flat)

    (A1,) = _tc_call(_mix_body, N, R_A, [AA1, ps1[0], ps1[1]],
                     [L1['W_A'][:F], L1['W_A'][F:], row(L1['b_A'])], [F])

    (PA2,) = _tc_call(_pair_post_body, MP, R_PP, [t1, PP1, Xp],
                      [bd8(L1['W_P'][:F]), bd8(L1['W_P'][F:]), t8(L1['b_P']),
                       bd8(L2['W_PA'][:PD]), bd8(L2['W_PA'][PD:]),
                       t8(L2['b_PA'])], [8 * F])

    ps2 = _make_segsum(M, N, F)(PA2, split2d)

    (h,) = _tc_call(_atom_final_body, N, R_A,
                    [atom_features, A1, ps2[0], ps2[1]],
                    [L2['W_AA'][:AD], L2['W_AA'][AD:], row(L2['b_AA']),
                     L2['W_A'][:F], L2['W_A'][F:], row(L2['b_A']),
                     Wd[:AD], Wd[AD:AD + F], Wd[AD + F:], row(params['b_dense']),
                     row(scale), row(shift)], [GF])
    return h


# speedup vs baseline: 7.2218x; 1.0185x over previous
import functools

import jax
import jax.numpy as jnp
from jax import lax
from jax.experimental import pallas as pl
from jax.experimental.pallas import tpu as pltpu
from jax.experimental.pallas import tpu_sc as plsc

F32 = jnp.float32


def _atom_pre_body(x, waa, baa, wg, bg, aa_o, g_o):
    xv = x[...]
    aa_o[...] = jnp.tanh(jnp.dot(xv, waa[...], preferred_element_type=F32) + baa[...])
    g_o[...] = jnp.dot(xv, wg[...], preferred_element_type=F32) + bg[...]


def _pair_pre_body(x, wpa, bpa, wpp, bpp, pa_o, pp_o):
    xv = x[...]
    pa_o[...] = jnp.tanh(jnp.dot(xv, wpa[...], preferred_element_type=F32) + bpa[...])
    pp_o[...] = jnp.tanh(jnp.dot(xv, wpp[...], preferred_element_type=F32) + bpp[...])


def _mix_body(aa, ps0, ps1, wa, wb, b, a_o):
    s = ps0[...] + ps1[...]
    a_o[...] = jnp.tanh(jnp.dot(aa[...], wa[...], preferred_element_type=F32)
                        + jnp.dot(s, wb[...], preferred_element_type=F32) + b[...])


def _pair_post_body(t, pp, x, wpa_t, wpb, bp, w2a, w2b, b2, pa2_o):
    p1 = jnp.tanh(jnp.dot(t[...], wpa_t[...], preferred_element_type=F32)
                  + jnp.dot(pp[...], wpb[...], preferred_element_type=F32) + bp[...])
    pa2_o[...] = jnp.tanh(jnp.dot(x[...], w2a[...], preferred_element_type=F32)
                          + jnp.dot(p1, w2b[...], preferred_element_type=F32) + b2[...])


def _atom_final_body(x, a1, ps0, ps1, waa2a, waa2b, baa2, wa2a, wa2b, ba2,
                     wda, wdb, wdc, bd, scale, shift, h_o):
    xv = x[...]
    a1v = a1[...]
    aa2 = jnp.tanh(jnp.dot(xv, waa2a[...], preferred_element_type=F32)
                   + jnp.dot(a1v, waa2b[...], preferred_element_type=F32) + baa2[...])
    pasum2 = ps0[...] + ps1[...]
    a2 = jnp.tanh(jnp.dot(aa2, wa2a[...], preferred_element_type=F32)
                  + jnp.dot(pasum2, wa2b[...], preferred_element_type=F32) + ba2[...])
    z = jnp.tanh(jnp.dot(xv, wda[...], preferred_element_type=F32)
                 + jnp.dot(a1v, wdb[...], preferred_element_type=F32)
                 + jnp.dot(a2, wdc[...], preferred_element_type=F32) + bd[...])
    h_o[...] = z * scale[...] + shift[...]


def _row_spec(r, d):
    return pl.BlockSpec((r, d), lambda i: (i, 0))


def _full_spec(shape):
    return pl.BlockSpec(shape, lambda i: tuple(0 for _ in shape))


def _tc_call(body, n_rows, block_rows, xs, weights, out_dims):
    grid = (n_rows // block_rows,)
    in_specs = ([_row_spec(block_rows, x.shape[1]) for x in xs]
                + [_full_spec(w.shape) for w in weights])
    out_specs = [_row_spec(block_rows, d) for d in out_dims]
    out_shape = [jax.ShapeDtypeStruct((n_rows, d), F32) for d in out_dims]
    outs = pl.pallas_call(
        body, grid=grid, in_specs=in_specs, out_specs=out_specs,
        out_shape=out_shape,
    )(*xs, *weights)
    return outs


def _sc_tanh(x):
    return 1.0 - 2.0 / (jnp.exp(2.0 * x) + 1.0)


_SC_PARAMS = pltpu.CompilerParams(use_tc_tiling_on_sc=False,
                                  needs_layout_passes=False)


def _make_segsum(M, N, F):
    info = plsc.get_sparse_core_info()
    NC, NS = info.num_cores, info.num_subcores
    NW = NC * NS
    per_tile = M // NW
    CI = 125
    KI = 8
    CHUNK = CI * KI
    PK = CHUNK // 8
    n_chunks = per_tile // CHUNK
    assert per_tile % CHUNK == 0 and per_tile % 8 == 0
    ZR = 1000
    n_zch = N // ZR
    assert N % ZR == 0
    zrounds = (n_zch + NS - 1) // NS
    mesh = plsc.VectorSubcoreMesh(core_axis_name="c", subcore_axis_name="s")

    @functools.partial(
        pl.kernel,
        out_type=jax.ShapeDtypeStruct((NC, N, F), F32),
        mesh=mesh,
        compiler_params=_SC_PARAMS,
        scratch_types=[
            pltpu.VMEM((KI, CI), jnp.int32),
            pltpu.VMEM((PK, 8 * F), F32),
            pltpu.VMEM((CHUNK, F), F32),
            pltpu.VMEM((ZR, F), F32),
            pltpu.VMEM_SHARED((N, F), F32),
            pltpu.SemaphoreType.DMA,
        ],
    )
    def segsum(pa_hbm, split_hbm, out_hbm, idx_v, pa_pk, pa_v, zbuf, table, sem):
        c = lax.axis_index("c")
        s = lax.axis_index("s")
        wid = s * NC + c

        def zb(r, carry):
            zbuf[r, :] = jnp.zeros((16,), F32)
            return carry
        lax.fori_loop(0, ZR, zb, None)
        for k in range(zrounds):
            cid = s + k * NS
            @pl.when(cid < n_zch)
            def _():
                pltpu.sync_copy(zbuf, table.at[pl.ds(cid * ZR, ZR)])
        plsc.subcore_barrier()

        base = wid * per_tile

        def chunk_body(ci, carry):
            off = base + ci * CHUNK
            pltpu.sync_copy(split_hbm.at[pl.ds(off // CI, KI)], idx_v)
            pltpu.sync_copy(pa_hbm.at[pl.ds(off // 8, PK)], pa_pk)

            def repack(r, carry2):
                for cc in range(8):
                    pa_v[8 * r + cc, :] = pa_pk[r, pl.ds(16 * cc, 16)]
                return carry2
            lax.fori_loop(0, PK, repack, None)

            descs = []
            for k in range(KI):
                descs.append(pltpu.async_copy(
                    pa_v.at[pl.ds(k * CI, CI)], table.at[idx_v.at[k]], sem,
                    add=True))
            for d in descs:
                d.wait()
            return carry
        lax.fori_loop(0, n_chunks, chunk_body, None)
        plsc.subcore_barrier()

        for k in range(zrounds):
            cid = s + k * NS
            @pl.when(cid < n_zch)
            def _():
                pltpu.sync_copy(table.at[pl.ds(cid * ZR, ZR)], zbuf)
                pltpu.sync_copy(zbuf, out_hbm.at[c, pl.ds(cid * ZR, ZR)])

    return segsum


def _make_gather_combine(M, N, F):
    info = plsc.get_sparse_core_info()
    NC, NS = info.num_cores, info.num_subcores
    NW = NC * NS
    per_tile = M // NW
    CP = 1000
    PK = CP // 8
    n_chunks = per_tile // CP
    assert per_tile % CP == 0
    pieces = []
    o = 0
    while o < CP:
        sz = min(128, CP - o)
        pieces.append((o, sz))
        o += sz
    mesh = plsc.VectorSubcoreMesh(core_axis_name="c", subcore_axis_name="s")

    @functools.partial(
        pl.kernel,
        out_type=jax.ShapeDtypeStruct((M // 8, 8 * F), F32),
        mesh=mesh,
        compiler_params=_SC_PARAMS,
        scratch_types=[
            pltpu.VMEM((CP,), jnp.int32),
            pltpu.VMEM((CP,), jnp.int32),
            pltpu.VMEM((CP, 2 * F), F32),
            pltpu.VMEM((CP, 2 * F), F32),
            pltpu.VMEM((PK, 8 * F), F32),
            pltpu.SemaphoreType.DMA,
        ],
    )
    def gather_combine(g_hbm, a2p_hbm, t_hbm, idx_i, idx_j, rows_i, rows_j,
                       t_v, sem):
        c = lax.axis_index("c")
        s = lax.axis_index("s")
        wid = s * NC + c
        pbase = wid * per_tile

        def chunk_body(ci, carry):
            p0 = pbase + ci * CP
            pltpu.sync_copy(a2p_hbm.at[0, pl.ds(p0, CP)], idx_i)
            pltpu.sync_copy(a2p_hbm.at[1, pl.ds(p0, CP)], idx_j)
            descs = []
            for (o, sz) in pieces:
                descs.append(pltpu.async_copy(
                    g_hbm.at[idx_i.at[pl.ds(o, sz)]],
                    rows_i.at[pl.ds(o, sz)], sem))
                descs.append(pltpu.async_copy(
                    g_hbm.at[idx_j.at[pl.ds(o, sz)]],
                    rows_j.at[pl.ds(o, sz)], sem))
            for d in descs:
                d.wait()

            def pair_body(p, carry2):
                u = rows_i[p, pl.ds(0, F)] + rows_j[p, pl.ds(F, F)]
                v = rows_j[p, pl.ds(0, F)] + rows_i[p, pl.ds(F, F)]
                t_v[p // 8, pl.ds((p % 8) * F, F)] = _sc_tanh(u) + _sc_tanh(v)
                return carry2
            lax.fori_loop(0, CP, pair_body, None)
            pltpu.sync_copy(t_v, t_hbm.at[pl.ds(p0 // 8, PK)])
            return carry
        lax.fori_loop(0, n_chunks, chunk_body, None)

    return gather_combine


def kernel(atom_features, pair_features, pair_split, atom_split, atom_to_pair, params):
    L1, L2 = params['layers']
    N, AD = atom_features.shape
    M, PD = pair_features.shape
    F = L1['W_AA'].shape[1]
    GF = params['W_dense'].shape[1]
    MP = M // 8

    row = lambda b: b[None, :]
    eye8 = jnp.eye(8, dtype=F32)
    fm = lambda w: jnp.einsum('fk,cd->fcdk', w, eye8).reshape(w.shape[0] * 8,
                                                             8 * w.shape[1])
    bd8 = lambda w: jax.scipy.linalg.block_diag(*([w] * 8))
    t8 = lambda b: jnp.tile(b, 8)[None, :]
    Wg = jnp.concatenate([L1['W_AP'][:AD], L1['W_AP'][AD:]], axis=1)
    bg = jnp.concatenate([L1['b_AP'], jnp.zeros_like(L1['b_AP'])])[None, :]
    Wd = params['W_dense']
    scale = params['bn_gamma'] / jnp.sqrt(params['bn_var'] + 1e-3)
    shift = params['bn_beta'] - params['bn_mean'] * scale

    a2pT = atom_to_pair.T
    split2d = pair_split.reshape(M // 125, 125)
    Xp2 = (pair_features.T.reshape(PD, MP, 8)
           .transpose(1, 0, 2).reshape(MP, PD * 8))

    R_A, R_PP = 5000, 2000

    AA1, G = _tc_call(_atom_pre_body, N, R_A, [atom_features],
                      [L1['W_AA'], row(L1['b_AA']), Wg, bg], [F, 2 * F])

    PA1, PP1 = _tc_call(_pair_pre_body, MP, R_PP, [Xp2],
                        [fm(L1['W_PA']), t8(L1['b_PA']),
                         fm(L1['W_PP']), t8(L1['b_PP'])],
                        [8 * F, 8 * F])

    ps1 = _make_segsum(M, N, F)(PA1, split2d)
    t1 = _make_gather_combine(M, N, F)(G, a2pT)

    (A1,) = _tc_call(_mix_body, N, R_A, [AA1, ps1[0], ps1[1]],
                     [L1['W_A'][:F], L1['W_A'][F:], row(L1['b_A'])], [F])

    (PA2,) = _tc_call(_pair_post_body, MP, R_PP, [t1, PP1, Xp2],
                      [bd8(L1['W_P'][:F]), bd8(L1['W_P'][F:]), t8(L1['b_P']),
                       fm(L2['W_PA'][:PD]), bd8(L2['W_PA'][PD:]),
                       t8(L2['b_PA'])], [8 * F])

    ps2 = _make_segsum(M, N, F)(PA2, split2d)

    (h,) = _tc_call(_atom_final_body, N, R_A,
                    [atom_features, A1, ps2[0], ps2[1]],
                    [L2['W_AA'][:AD], L2['W_AA'][AD:], row(L2['b_AA']),
                     L2['W_A'][:F], L2['W_A'][F:], row(L2['b_A']),
                     Wd[:AD], Wd[AD:AD + F], Wd[AD + F:], row(params['b_dense']),
                     row(scale), row(shift)], [GF])
    return h
